# baseline (speedup 1.0000x reference)
import functools

import jax
import jax.numpy as jnp
from jax import lax
from jax.experimental import pallas as pl
from jax.experimental.pallas import tpu as pltpu
from jax.experimental.pallas import tpu_sc as plsc

N = 10000
E = 320000
NC, NS = 2, 16
NW = NC * NS
EPW = E // NW
C = 125
NCHUNK = EPW // C
RB = 624
REM = N - RB * NS
RZ = RB + REM
F1, F2, F3 = 128, 64, 64
CW = 16

_SC_PARAMS = pltpu.CompilerParams(use_tc_tiling_on_sc=False)


def _mesh():
    return plsc.VectorSubcoreMesh(core_axis_name="c", subcore_axis_name="s")


def _zero_acc(sid, z_h, acc):
    pltpu.sync_copy(z_h.at[pl.ds(0, RB)], acc.at[pl.ds(sid * RB, RB)])

    @pl.when(sid == 0)
    def _():
        pltpu.sync_copy(z_h.at[pl.ds(0, REM)], acc.at[pl.ds(RB * NS, REM)])


def _copy_out(sid, acc, out2d):
    rows = pl.ds(sid * RB, RB)
    pltpu.sync_copy(acc.at[rows], out2d.at[rows])

    @pl.when(sid == 0)
    def _():
        tail = pl.ds(RB * NS, REM)
        pltpu.sync_copy(acc.at[tail], out2d.at[tail])


def _sc_degrees(src3, dst3, ones_h, zeros_h):
    @functools.partial(
        pl.kernel,
        out_type=(jax.ShapeDtypeStruct((NC, N, CW), jnp.float32),
                  jax.ShapeDtypeStruct((NC, N, CW), jnp.float32)),
        mesh=_mesh(),
        compiler_params=_SC_PARAMS,
        scratch_types=[
            pltpu.VMEM((NCHUNK, C), jnp.int32),
            pltpu.VMEM((NCHUNK, C), jnp.int32),
            pltpu.VMEM((C, CW), jnp.float32),
            pltpu.VMEM_SHARED((N, CW), jnp.float32),
            pltpu.VMEM_SHARED((N, CW), jnp.float32),
            pltpu.SemaphoreType.DMA,
        ],
    )
    def k(src_h, dst_h, ones_hr, z_h, dego_h, degi_h,
          src_v, dst_v, ones_v, dego_s, degi_s, ssem):
        cid = lax.axis_index("c")
        sid = lax.axis_index("s")
        wid = sid * NC + cid
        _zero_acc(sid, z_h, dego_s)
        _zero_acc(sid, z_h, degi_s)
        pltpu.sync_copy(ones_hr, ones_v)
        pltpu.sync_copy(src_h.at[wid], src_v)
        pltpu.sync_copy(dst_h.at[wid], dst_v)
        plsc.subcore_barrier()

        @pl.loop(0, NCHUNK, step=4)
        def _(j):
            for o in range(4):
                pltpu.async_copy(ones_v, dego_s.at[src_v.at[j + o]], ssem,
                                 add=True)
                pltpu.async_copy(ones_v, degi_s.at[dst_v.at[j + o]], ssem,
                                 add=True)
            for o in range(8):
                pltpu.make_async_copy(ones_v, dego_s.at[src_v.at[0]],
                                      ssem).wait()

        plsc.subcore_barrier()
        _copy_out(sid, dego_s, dego_h.at[cid])
        _copy_out(sid, degi_s, degi_h.at[cid])

    return k(src3, dst3, ones_h, zeros_h)


NCHUNK2 = (E // NS) // C


def _sc_propagate2(tables, gidx2, sidx2, zeros_list, Ds, K, swaps, splits):
    NT = len(tables)
    NB = NCHUNK2 // K
    assert NCHUNK2 % K == 0 and NB % 2 == 0

    bufs_types = [pltpu.VMEM((C, D), jnp.float32)
                  for _ in range(2) for D in Ds for _k in range(K)]
    acc_types = [pltpu.VMEM_SHARED((N, D), jnp.float32) for D in Ds]
    sem_types = [pltpu.SemaphoreType.DMA] * (4 * NT)

    @functools.partial(
        pl.kernel,
        out_type=tuple(jax.ShapeDtypeStruct((NC, N, D), jnp.float32) for D in Ds),
        mesh=_mesh(),
        compiler_params=_SC_PARAMS,
        scratch_types=[
            pltpu.VMEM((NCHUNK2, C), jnp.int32),
            pltpu.VMEM((NCHUNK2, C), jnp.int32),
        ] + bufs_types + acc_types + sem_types,
    )
    def k(*refs):
        t_h = refs[:NT]
        g_h, s_h = refs[NT], refs[NT + 1]
        z_h = refs[NT + 2:NT + 2 + NT]
        agg_h = refs[NT + 2 + NT:NT + 2 + 2 * NT]
        g_v, s_v = refs[3 * NT + 2], refs[3 * NT + 3]
        p = 3 * NT + 4
        bufs = [[[refs[p + (g * NT + i) * K + k_] for k_ in range(K)]
                 for i in range(NT)] for g in range(2)]
        p += 2 * NT * K
        accs = refs[p:p + NT]
        p += NT
        gsem = [[refs[p + g * NT + i] for i in range(NT)] for g in range(2)]
        p += 2 * NT
        ssem = [[refs[p + g * NT + i] for i in range(NT)] for g in range(2)]

        cid = lax.axis_index("c")
        sid = lax.axis_index("s")
        for i in range(NT):
            _zero_acc(sid, z_h[i], accs[i])
        pltpu.sync_copy(g_h.at[sid], g_v)
        pltpu.sync_copy(s_h.at[sid], s_v)
        plsc.subcore_barrier()

        def table(i):
            return t_h[i].at[cid] if splits[i] else t_h[i]

        def gv(i):
            return s_v if swaps[i] else g_v

        def sv(i):
            return g_v if swaps[i] else s_v

        def gather(i, j, grp, slot):
            pltpu.async_copy(table(i).at[gv(i).at[j]], bufs[grp][i][slot],
                             gsem[grp][i])

        def scatter(i, j, grp, slot):
            pltpu.async_copy(bufs[grp][i][slot], accs[i].at[sv(i).at[j]],
                             ssem[grp][i], add=True)

        def wait_g(grp, i):
            pltpu.make_async_copy(table(i).at[gv(i).at[0]], bufs[grp][i][0],
                                  gsem[grp][i]).wait()

        def wait_s(grp, i):
            pltpu.make_async_copy(bufs[grp][i][0], accs[i].at[sv(i).at[0]],
                                  ssem[grp][i]).wait()

        for k_ in range(K):
            for i in range(NT):
                gather(i, k_, 0, k_)
        for k_ in range(K):
            for i in range(NT):
                gather(i, K + k_, 1, k_)

        @pl.loop(0, NB, step=2)
        def _(b):
            base = b * K
            for k_ in range(K):
                for i in range(NT):
                    wait_g(0, i)
            for k_ in range(K):
                for i in range(NT):
                    scatter(i, base + k_, 0, k_)
            for k_ in range(K):
                for i in range(NT):
                    wait_g(1, i)
            for k_ in range(K):
                for i in range(NT):
                    scatter(i, base + K + k_, 1, k_)

            @pl.when(b + 2 < NB)
            def _():
                for k_ in range(K):
                    for i in range(NT):
                        wait_s(0, i)
                for k_ in range(K):
                    for i in range(NT):
                        gather(i, base + 2 * K + k_, 0, k_)
                for k_ in range(K):
                    for i in range(NT):
                        wait_s(1, i)
                for k_ in range(K):
                    for i in range(NT):
                        gather(i, base + 3 * K + k_, 1, k_)

        for k_ in range(K):
            for i in range(NT):
                wait_s(0, i)
                wait_s(1, i)
        plsc.subcore_barrier()
        for i in range(NT):
            _copy_out(sid, accs[i], agg_h[i].at[cid])

    return k(*tables, gidx2, sidx2, *zeros_list)


def _sc_layer1(t1pair, src2, dst2, z64):
    return _sc_propagate2([t1pair], src2, dst2, [z64], [F2], 2,
                          [False], [True])[0]


def _sc_layer2c(t2pair, tnd, src2, dst2, z32, z16):
    return _sc_propagate2([t2pair, tnd], src2, dst2, [z32, z16],
                          [F2 // 2, CW], 4, [False, True], [True, False])


def _norms(do_ref, di_ref):
    dego = do_ref[0, :, 0:1] + do_ref[1, :, 0:1]
    degi = di_ref[0, :, 0:1] + di_ref[1, :, 0:1]
    ns = jnp.where(dego > 0, lax.rsqrt(dego), 0.0)
    nd = jnp.where(degi > 0, lax.rsqrt(degi), 0.0)
    return ns, nd


def _dot(a, b):
    ah = a.astype(jnp.bfloat16)
    al = (a - ah.astype(jnp.float32)).astype(jnp.bfloat16)
    bh = b.astype(jnp.bfloat16)
    bl = (b - bh.astype(jnp.float32)).astype(jnp.bfloat16)

    def d(u, v):
        return lax.dot_general(u, v, (((1,), (0,)), ((), ())),
                               preferred_element_type=jnp.float32)

    return d(ah, bh) + d(ah, bl) + d(al, bh)


G = 10
BR = N // G


def _bs(shape, im):
    return pl.BlockSpec(shape, im)


def _row(i):
    return (i, 0)


def _prow(i):
    return (0, i, 0)


def _full(i):
    return (0, 0)


def _tc_prep(x, W1a, W1b, dego_p, degi_p):
    def body(x_ref, wa_ref, wb_ref, do_ref, di_ref, t1p_ref, tnd_ref):
        ns, nd = _norms(do_ref, di_ref)
        xs = x_ref[...] * ns
        t1p_ref[0] = _dot(xs, wa_ref[...])
        t1p_ref[1] = _dot(xs, wb_ref[...])
        tnd_ref[...] = jnp.broadcast_to(nd, (BR, CW))

    return pl.pallas_call(
        body,
        grid=(G,),
        in_specs=[_bs((BR, F1), _row), _bs((F1, F2), _full), _bs((F1, F2), _full),
                  _bs((NC, BR, CW), _prow), _bs((NC, BR, CW), _prow)],
        out_specs=(_bs((NC, BR, F2), _prow), _bs((BR, CW), _row)),
        out_shape=(jax.ShapeDtypeStruct((NC, N, F2), jnp.float32),
                   jax.ShapeDtypeStruct((N, CW), jnp.float32)),
    )(x, W1a, W1b, dego_p, degi_p)


def _tc_mid(agg1, dego_p, degi_p, b1ar, b1br, W2q):
    def body(a_ref, do_ref, di_ref, ba_ref, bb_ref, waa_ref, wba_ref,
             wab_ref, wbb_ref, t2p_ref):
        ns, nd = _norms(do_ref, di_ref)
        h1a = jnp.maximum(a_ref[0] * nd + ba_ref[...], 0.0) * ns
        h1b = jnp.maximum(a_ref[1] * nd + bb_ref[...], 0.0) * ns
        t2p_ref[0] = _dot(h1a, waa_ref[...]) + _dot(h1b, wba_ref[...])
        t2p_ref[1] = _dot(h1a, wab_ref[...]) + _dot(h1b, wbb_ref[...])

    H = F2 // 2
    return pl.pallas_call(
        body,
        grid=(G,),
        in_specs=[_bs((NC, BR, F2), _prow),
                  _bs((NC, BR, CW), _prow), _bs((NC, BR, CW), _prow),
                  _bs((1, F2), _full), _bs((1, F2), _full),
                  _bs((F2, H), _full), _bs((F2, H), _full),
                  _bs((F2, H), _full), _bs((F2, H), _full)],
        out_specs=_bs((NC, BR, H), _prow),
        out_shape=jax.ShapeDtypeStruct((NC, N, H), jnp.float32),
    )(agg1, dego_p, degi_p, b1ar, b1br, *W2q)


def _tc_final(agg2, c_p, dego_p, degi_p, b2ar, b2br, W3a, W3b, b3r):
    H = F2 // 2

    def body(a_ref, c_ref, do_ref, di_ref, b2a_ref, b2b_ref, wa_ref, wb_ref,
             b3_ref, o_ref, acca_ref, accb_ref):
        i = pl.program_id(0)

        @pl.when(i == 0)
        def _():
            acca_ref[...] = jnp.zeros((1, H), jnp.float32)
            accb_ref[...] = jnp.zeros((1, H), jnp.float32)

        ns, nd = _norms(do_ref, di_ref)
        h2a = jnp.maximum(a_ref[0] * nd + b2a_ref[...], 0.0)
        h2b = jnp.maximum(a_ref[1] * nd + b2b_ref[...], 0.0)
        wv = c_ref[0, :, 0:1] * ns * (1.0 / N)
        acca_ref[...] += jnp.sum(h2a * wv, axis=0, keepdims=True)
        accb_ref[...] += jnp.sum(h2b * wv, axis=0, keepdims=True)

        @pl.when(i == G - 1)
        def _():
            o_ref[...] = (_dot(acca_ref[...], wa_ref[...])
                          + _dot(accb_ref[...], wb_ref[...]) + b3_ref[...])

    return pl.pallas_call(
        body,
        grid=(G,),
        in_specs=[_bs((NC, BR, H), _prow), _bs((NC, BR, CW), _prow),
                  _bs((NC, BR, CW), _prow), _bs((NC, BR, CW), _prow),
                  _bs((1, H), _full), _bs((1, H), _full),
                  _bs((H, F3), _full), _bs((H, F3), _full), _bs((1, F3), _full)],
        out_specs=_bs((1, F3), _full),
        out_shape=jax.ShapeDtypeStruct((1, F3), jnp.float32),
        scratch_shapes=[pltpu.VMEM((1, H), jnp.float32),
                        pltpu.VMEM((1, H), jnp.float32)],
    )(agg2, c_p, dego_p, degi_p, b2ar, b2br, W3a, W3b, b3r)


def kernel(x, edge_index, W1, b1, W2, b2, W3, b3):
    src3 = edge_index[0].reshape(NW, NCHUNK, C)
    dst3 = edge_index[1].reshape(NW, NCHUNK, C)
    src2 = edge_index[0].reshape(NS, NCHUNK2, C)
    dst2 = edge_index[1].reshape(NS, NCHUNK2, C)
    ones16 = jnp.ones((C, CW), jnp.float32)
    z16 = jnp.zeros((RZ, CW), jnp.float32)
    z32 = jnp.zeros((RZ, F2 // 2), jnp.float32)
    z64 = jnp.zeros((RZ, F2), jnp.float32)
    H = F2 // 2

    dego_p, degi_p = _sc_degrees(src3, dst3, ones16, z16)
    t1pair, tnd = _tc_prep(x, W1[:, :F2], W1[:, F2:], dego_p, degi_p)
    agg1 = _sc_layer1(t1pair, src2, dst2, z64)
    t2pair = _tc_mid(agg1, dego_p, degi_p,
                     b1[:F2].reshape(1, F2), b1[F2:].reshape(1, F2),
                     (W2[:F2, :H], W2[F2:, :H], W2[:F2, H:], W2[F2:, H:]))
    agg2, c_p = _sc_layer2c(t2pair, tnd, src2, dst2, z32, z16)
    return _tc_final(agg2, c_p, dego_p, degi_p,
                     b2[:H].reshape(1, H), b2[H:].reshape(1, H),
                     W3[:H], W3[H:], b3.reshape(1, F3))
